# Initial kernel scaffold; baseline (speedup 1.0000x reference)
import jax
import jax.numpy as jnp
from jax.experimental import pallas as pl


def kernel(states, arc_labels, arc_weights, arc_to, backoff_weights, backoff_to, unk_prob):
    raise NotImplementedError("write your pallas kernel here")


# speedup vs baseline: 10.6080x; 10.6080x over previous
import functools

import jax
import jax.numpy as jnp
from jax import lax
from jax.experimental import pallas as pl
from jax.experimental.pallas import tpu as pltpu
from jax.experimental.pallas import tpu_sc as plsc

V = 1024
A = 16
LVL = 4
B = 16384

NW = 32
PER_W = B // NW
PIECE = 128
NPIECE = PER_W // PIECE

BB = 256
ACCW = 8


def _sc_gather_body(states, arc_labels, arc_weights, arc_to, backoff_w,
                    backoff_to, lbl_out, w_out, to_out, acc_out,
                    cur_v, lbl_v, w_v, to_v, bw_v, accout_v,
                    gsem, osem):
    nc = plsc.get_sparse_core_info().num_cores
    c = lax.axis_index("c")
    s = lax.axis_index("s")
    wid = s * nc + c
    base = wid * PER_W

    pltpu.sync_copy(states.at[pl.ds(base, PER_W)], cur_v.at[pl.ds(0, PER_W)])

    out_handles = []
    for l in range(LVL):
        hs = []
        for j in range(NPIECE):
            idx = cur_v.at[pl.ds(l * PER_W + j * PIECE, PIECE)]
            dst = pl.ds(j * PIECE, PIECE)
            hs.append(pltpu.async_copy(arc_labels.at[idx], lbl_v.at[l, dst], gsem))
            hs.append(pltpu.async_copy(arc_weights.at[idx], w_v.at[l, dst], gsem))
            hs.append(pltpu.async_copy(arc_to.at[idx], to_v.at[l, dst], gsem))
            hs.append(pltpu.async_copy(
                backoff_w.at[idx],
                bw_v.at[pl.ds(l * PER_W + j * PIECE, PIECE)], gsem))
            if l + 1 < LVL:
                hs.append(pltpu.async_copy(
                    backoff_to.at[idx],
                    cur_v.at[pl.ds((l + 1) * PER_W + j * PIECE, PIECE)], gsem))
        for h in hs:
            h.wait()
        out_handles.append(pltpu.async_copy(
            lbl_v.at[l], lbl_out.at[l, pl.ds(base, PER_W)], osem))
        out_handles.append(pltpu.async_copy(
            w_v.at[l], w_out.at[l, pl.ds(base, PER_W)], osem))
        out_handles.append(pltpu.async_copy(
            to_v.at[l], to_out.at[l, pl.ds(base, PER_W)], osem))
        for chunk in range(PER_W // 16):
            off = chunk * 16
            if l == 0:
                prev = jnp.zeros((16,), jnp.float32)
                accout_v[pl.ds(off, 16)] = prev
            else:
                prev = accout_v[pl.ds(l * PER_W + off, 16)]
            accout_v[pl.ds((l + 1) * PER_W + off, 16)] = (
                prev + bw_v[pl.ds(l * PER_W + off, 16)])
    for l in range(LVL + 1):
        out_handles.append(pltpu.async_copy(
            accout_v.at[pl.ds(l * PER_W, PER_W)],
            acc_out.at[l, pl.ds(base, PER_W)], osem))
    for l in range(LVL + 1, ACCW):
        out_handles.append(pltpu.async_copy(
            accout_v.at[pl.ds(0, PER_W)],
            acc_out.at[l, pl.ds(base, PER_W)], osem))
    for h in out_handles:
        h.wait()


@functools.cache
def _build_sc_gather():
    return pl.kernel(
        _sc_gather_body,
        mesh=plsc.VectorSubcoreMesh(core_axis_name="c", subcore_axis_name="s"),
        compiler_params=pltpu.CompilerParams(use_tc_tiling_on_sc=False),
        out_type=[
            jax.ShapeDtypeStruct((LVL, B, A), jnp.int32),
            jax.ShapeDtypeStruct((LVL, B, A), jnp.float32),
            jax.ShapeDtypeStruct((LVL, B, A), jnp.int32),
            jax.ShapeDtypeStruct((ACCW, B), jnp.float32),
        ],
        scratch_types=[
            pltpu.VMEM((LVL * PER_W,), jnp.int32),
            pltpu.VMEM((LVL, PER_W, A), jnp.int32),
            pltpu.VMEM((LVL, PER_W, A), jnp.float32),
            pltpu.VMEM((LVL, PER_W, A), jnp.int32),
            pltpu.VMEM((LVL * PER_W,), jnp.float32),
            pltpu.VMEM(((LVL + 1) * PER_W,), jnp.float32),
            pltpu.SemaphoreType.DMA,
            pltpu.SemaphoreType.DMA,
        ],
    )


def _tc_expand_body(unk_ref, lbl_ref, w_ref, to_ref, acc_ref,
                    scores_ref, nxt_ref):
    vv = lax.broadcasted_iota(jnp.int32, (BB, V), 1)
    unk = unk_ref[0, 0]
    acct = jnp.transpose(acc_ref[...])
    scores = jnp.broadcast_to(acct[:, LVL:LVL + 1] + unk, (BB, V))
    nxt = jnp.zeros((BB, V), jnp.int32)
    for a in range(LVL * A - 1, -1, -1):
        l, k = divmod(a, A)
        m = vv == lbl_ref[l, :, k:k + 1]
        cand = acct[:, l:l + 1] + w_ref[l, :, k:k + 1]
        scores = jnp.where(m, cand, scores)
        nxt = jnp.where(m, to_ref[l, :, k:k + 1], nxt)
    scores_ref[...] = scores
    nxt_ref[...] = nxt


_tc_expand = pl.pallas_call(
    _tc_expand_body,
    grid=(B // BB,),
    in_specs=[
        pl.BlockSpec(memory_space=pltpu.SMEM),
        pl.BlockSpec((LVL, BB, A), lambda i: (0, i, 0)),
        pl.BlockSpec((LVL, BB, A), lambda i: (0, i, 0)),
        pl.BlockSpec((LVL, BB, A), lambda i: (0, i, 0)),
        pl.BlockSpec((ACCW, BB), lambda i: (0, i)),
    ],
    out_specs=[
        pl.BlockSpec((BB, V), lambda i: (i, 0)),
        pl.BlockSpec((BB, V), lambda i: (i, 0)),
    ],
    out_shape=[
        jax.ShapeDtypeStruct((B, V), jnp.float32),
        jax.ShapeDtypeStruct((B, V), jnp.int32),
    ],
)


def kernel(states, arc_labels, arc_weights, arc_to, backoff_weights,
           backoff_to, unk_prob):
    lbl, w, to, acc = _build_sc_gather()(
        states, arc_labels, arc_weights, arc_to, backoff_weights, backoff_to)
    unk2 = unk_prob.reshape(1, 1)
    scores, nxt = _tc_expand(unk2, lbl, w, to, acc)
    return scores, nxt


# speedup vs baseline: 31.3229x; 2.9528x over previous
import functools

import jax
import jax.numpy as jnp
from jax import lax
from jax.experimental import pallas as pl
from jax.experimental.pallas import tpu as pltpu
from jax.experimental.pallas import tpu_sc as plsc

V = 1024
A = 16
LVL = 4
B = 16384

NW = 32
PER_W = B // NW
PIECE = 128
NPIECE = PER_W // PIECE

K = 4
NCHUNK = PER_W // K


def _sc_body(states, arc_labels, arc_weights, arc_to, backoff_w,
             backoff_to, unk, scores_out, nxt_out,
             cur_v, lbl_v, w_v, to_v, bw_v, acc_v, unk_v,
             rs_v, rn_v, gsem, osem):
    nc = plsc.get_sparse_core_info().num_cores
    c = lax.axis_index("c")
    s = lax.axis_index("s")
    wid = s * nc + c
    base = wid * PER_W

    pltpu.sync_copy(states.at[pl.ds(base, PER_W)], cur_v.at[pl.ds(0, PER_W)])
    pltpu.sync_copy(unk, unk_v.at[pl.ds(0, 1)])

    for l in range(LVL):
        hs = []
        for j in range(NPIECE):
            idx = cur_v.at[pl.ds(l * PER_W + j * PIECE, PIECE)]
            dst = pl.ds(j * PIECE, PIECE)
            hs.append(pltpu.async_copy(arc_labels.at[idx], lbl_v.at[l, dst], gsem))
            hs.append(pltpu.async_copy(arc_weights.at[idx], w_v.at[l, dst], gsem))
            hs.append(pltpu.async_copy(arc_to.at[idx], to_v.at[l, dst], gsem))
            hs.append(pltpu.async_copy(
                backoff_w.at[idx],
                bw_v.at[pl.ds(l * PER_W + j * PIECE, PIECE)], gsem))
            if l + 1 < LVL:
                hs.append(pltpu.async_copy(
                    backoff_to.at[idx],
                    cur_v.at[pl.ds((l + 1) * PER_W + j * PIECE, PIECE)], gsem))
        for h in hs:
            h.wait()
        for chunk in range(PER_W // 16):
            off = chunk * 16
            if l == 0:
                prev = jnp.zeros((16,), jnp.float32)
                acc_v[pl.ds(off, 16)] = prev
            else:
                prev = acc_v[pl.ds(l * PER_W + off, 16)]
            acc_v[pl.ds((l + 1) * PER_W + off, 16)] = (
                prev + bw_v[pl.ds(l * PER_W + off, 16)])

    unk_s = unk_v[...][0]

    def drain(buf):
        pltpu.make_async_copy(
            scores_out.at[pl.ds(base, K)], rs_v.at[buf], osem).wait()
        pltpu.make_async_copy(
            nxt_out.at[pl.ds(base, K)], rn_v.at[buf], osem).wait()

    def do_chunk(cc, buf):
        row0 = cc * K
        rs = rs_v.at[buf]
        rn = rn_v.at[buf]
        for r in range(K):
            row = row0 + r
            bval = acc_v[pl.ds(LVL * PER_W + row, 16)][0] + unk_s
            bvec = jnp.broadcast_to(bval, (16,))
            zvec = jnp.zeros((16,), jnp.int32)
            for i in range(V // 16):
                rs[r, pl.ds(i * 16, 16)] = bvec
                rn[r, pl.ds(i * 16, 16)] = zvec
            for l in range(LVL - 1, -1, -1):
                lblv = lbl_v[l, row]
                wacc = w_v[l, row] + acc_v[pl.ds(l * PER_W + row, 16)][0]
                tov = to_v[l, row]
                m = lblv < V
                ridx = jnp.broadcast_to(jnp.int32(r), (16,))
                plsc.store_scatter(rs, [ridx, lblv], wacc, mask=m)
                plsc.store_scatter(rn, [ridx, lblv], tov, mask=m)
        pltpu.async_copy(rs, scores_out.at[pl.ds(base + row0, K)], osem)
        pltpu.async_copy(rn, nxt_out.at[pl.ds(base + row0, K)], osem)

    for cc in range(2):
        do_chunk(cc, cc)

    def loop_body(i, _):
        for buf in range(2):
            drain(buf)
            do_chunk(2 + 2 * i + buf, buf)
        return 0

    lax.fori_loop(0, (NCHUNK - 2) // 2, loop_body, 0)
    for buf in range(2):
        drain(buf)


@functools.cache
def _build_sc_kernel():
    return pl.kernel(
        _sc_body,
        mesh=plsc.VectorSubcoreMesh(core_axis_name="c", subcore_axis_name="s"),
        compiler_params=pltpu.CompilerParams(
            use_tc_tiling_on_sc=False, needs_layout_passes=False),
        out_type=[
            jax.ShapeDtypeStruct((B, V), jnp.float32),
            jax.ShapeDtypeStruct((B, V), jnp.int32),
        ],
        scratch_types=[
            pltpu.VMEM((LVL * PER_W,), jnp.int32),
            pltpu.VMEM((LVL, PER_W, A), jnp.int32),
            pltpu.VMEM((LVL, PER_W, A), jnp.float32),
            pltpu.VMEM((LVL, PER_W, A), jnp.int32),
            pltpu.VMEM((LVL * PER_W,), jnp.float32),
            pltpu.VMEM(((LVL + 1) * PER_W + 16,), jnp.float32),
            pltpu.VMEM((16,), jnp.float32),
            pltpu.VMEM((2, K, V), jnp.float32),
            pltpu.VMEM((2, K, V), jnp.int32),
            pltpu.SemaphoreType.DMA,
            pltpu.SemaphoreType.DMA,
        ],
    )


def kernel(states, arc_labels, arc_weights, arc_to, backoff_weights,
           backoff_to, unk_prob):
    scores, nxt = _build_sc_kernel()(
        states, arc_labels, arc_weights, arc_to, backoff_weights, backoff_to,
        unk_prob.reshape(1))
    return scores, nxt


# speedup vs baseline: 36.9732x; 1.1804x over previous
import functools

import jax
import jax.numpy as jnp
from jax import lax
from jax.experimental import pallas as pl
from jax.experimental.pallas import tpu as pltpu
from jax.experimental.pallas import tpu_sc as plsc

V = 1024
A = 16
LVL = 4
B = 16384

NW = 32
PER_W = B // NW
PIECE = 128
NPIECE = PER_W // PIECE

K = 4
NCHUNK = PER_W // K

NS_TOT = 250000
T_RANGE = 7816
T_CS = 512
T_NCH = 16


def _t_body(lblT, wT, toT, lbl_out, w_out, to_out,
            pb_l, pb_w, pb_t, rb_l, rb_w, rb_t, gsem0, gsem1, osem0, osem1):
    gsems = (gsem0, gsem1)
    osems = (osem0, osem1)
    nc = plsc.get_sparse_core_info().num_cores
    c = lax.axis_index("c")
    s = lax.axis_index("s")
    wid = s * nc + c
    r0 = jnp.minimum(wid * T_RANGE, NS_TOT - T_RANGE)
    iota16 = lax.iota(jnp.int32, 16)

    def s_of(k):
        return r0 + jnp.minimum(k * T_CS, T_RANGE - T_CS)

    def issue_in(k, par):
        s0 = s_of(k)
        for a in range(16):
            pltpu.async_copy(
                lblT.at[a, pl.ds(s0, T_CS)], pb_l.at[par, a], gsems[par])
            pltpu.async_copy(
                wT.at[a, pl.ds(s0, T_CS)], pb_w.at[par, a], gsems[par])
            pltpu.async_copy(
                toT.at[a, pl.ds(s0, T_CS)], pb_t.at[par, a], gsems[par])

    def wait_in(par):
        for a in range(16):
            pltpu.make_async_copy(
                lblT.at[0, pl.ds(0, T_CS)], pb_l.at[par, a], gsems[par]).wait()
            pltpu.make_async_copy(
                wT.at[0, pl.ds(0, T_CS)], pb_w.at[par, a], gsems[par]).wait()
            pltpu.make_async_copy(
                toT.at[0, pl.ds(0, T_CS)], pb_t.at[par, a], gsems[par]).wait()

    def drain_out(par):
        s0 = s_of(0)
        pltpu.make_async_copy(
            lbl_out.at[pl.ds(s0, T_CS)], rb_l.at[par], osems[par]).wait()
        pltpu.make_async_copy(
            w_out.at[pl.ds(s0, T_CS)], rb_w.at[par], osems[par]).wait()
        pltpu.make_async_copy(
            to_out.at[pl.ds(s0, T_CS)], rb_t.at[par], osems[par]).wait()

    def transpose_and_out(k, par):
        s0 = s_of(k)

        def g_body(g, _):
            ridx = g * 16 + iota16
            for a in range(16):
                cidx = jnp.broadcast_to(jnp.int32(a), (16,))
                plsc.store_scatter(
                    rb_l.at[par], [ridx, cidx],
                    pb_l[par, a, pl.ds(g * 16, 16)])
                plsc.store_scatter(
                    rb_w.at[par], [ridx, cidx],
                    pb_w[par, a, pl.ds(g * 16, 16)])
                plsc.store_scatter(
                    rb_t.at[par], [ridx, cidx],
                    pb_t[par, a, pl.ds(g * 16, 16)])
            return 0

        lax.fori_loop(0, T_CS // 16, g_body, 0)
        pltpu.async_copy(rb_l.at[par], lbl_out.at[pl.ds(s0, T_CS)], osems[par])
        pltpu.async_copy(rb_w.at[par], w_out.at[pl.ds(s0, T_CS)], osems[par])
        pltpu.async_copy(rb_t.at[par], to_out.at[pl.ds(s0, T_CS)], osems[par])

    issue_in(0, 0)
    issue_in(1, 1)
    wait_in(0)
    transpose_and_out(0, 0)
    issue_in(2, 0)
    wait_in(1)
    transpose_and_out(1, 1)
    issue_in(3, 1)

    def pair_body(p, _):
        for par in range(2):
            k = 2 + 2 * p + par
            wait_in(par)
            drain_out(par)
            transpose_and_out(k, par)
            issue_in(k + 2, par)
        return 0

    lax.fori_loop(0, (T_NCH - 4) // 2, pair_body, 0)
    for par in range(2):
        k = T_NCH - 2 + par
        wait_in(par)
        drain_out(par)
        transpose_and_out(k, par)
    for par in range(2):
        drain_out(par)


def _sc_body(states, arc_labels, arc_weights, arc_to, backoff_w,
             backoff_to, unk, scores_out, nxt_out,
             cur_v, lbl_v, w_v, to_v, bw_v, acc_v, unk_v,
             rs_v, rn_v, gsem, osem0, osem1):
    nc = plsc.get_sparse_core_info().num_cores
    c = lax.axis_index("c")
    s = lax.axis_index("s")
    wid = s * nc + c
    base = wid * PER_W

    pltpu.sync_copy(states.at[pl.ds(base, PER_W)], cur_v.at[pl.ds(0, PER_W)])
    pltpu.sync_copy(unk, unk_v.at[pl.ds(0, 1)])

    for l in range(LVL):
        hs = []
        for j in range(NPIECE):
            idx = cur_v.at[pl.ds(l * PER_W + j * PIECE, PIECE)]
            dst = pl.ds(j * PIECE, PIECE)
            hs.append(pltpu.async_copy(arc_labels.at[idx], lbl_v.at[l, dst], gsem))
            hs.append(pltpu.async_copy(arc_weights.at[idx], w_v.at[l, dst], gsem))
            hs.append(pltpu.async_copy(arc_to.at[idx], to_v.at[l, dst], gsem))
            hs.append(pltpu.async_copy(
                backoff_w.at[idx],
                bw_v.at[pl.ds(l * PER_W + j * PIECE, PIECE)], gsem))
            if l + 1 < LVL:
                hs.append(pltpu.async_copy(
                    backoff_to.at[idx],
                    cur_v.at[pl.ds((l + 1) * PER_W + j * PIECE, PIECE)], gsem))
        for h in hs:
            h.wait()
        for chunk in range(PER_W // 16):
            off = chunk * 16
            if l == 0:
                prev = jnp.zeros((16,), jnp.float32)
                acc_v[pl.ds(off, 16)] = prev
            else:
                prev = acc_v[pl.ds(l * PER_W + off, 16)]
            acc_v[pl.ds((l + 1) * PER_W + off, 16)] = (
                prev + bw_v[pl.ds(l * PER_W + off, 16)])

    unk_s = unk_v[...][0]

    osems = (osem0, osem1)

    def drain(buf):
        pltpu.make_async_copy(
            scores_out.at[pl.ds(base, K)], rs_v.at[buf], osems[buf]).wait()
        pltpu.make_async_copy(
            nxt_out.at[pl.ds(base, K)], rn_v.at[buf], osems[buf]).wait()

    def do_chunk(cc, buf):
        row0 = cc * K
        rs = rs_v.at[buf]
        rn = rn_v.at[buf]
        for r in range(K):
            row = row0 + r
            bval = acc_v[pl.ds(LVL * PER_W + row, 16)][0] + unk_s
            bvec = jnp.broadcast_to(bval, (16,))
            zvec = jnp.zeros((16,), jnp.int32)
            for i in range(V // 16):
                rs[r, pl.ds(i * 16, 16)] = bvec
                rn[r, pl.ds(i * 16, 16)] = zvec
            for l in range(LVL - 1, -1, -1):
                lblv = lbl_v[l, row]
                wacc = w_v[l, row] + acc_v[pl.ds(l * PER_W + row, 16)][0]
                tov = to_v[l, row]
                m = lblv < V
                ridx = jnp.broadcast_to(jnp.int32(r), (16,))
                plsc.store_scatter(rs, [ridx, lblv], wacc, mask=m)
                plsc.store_scatter(rn, [ridx, lblv], tov, mask=m)
        pltpu.async_copy(rs, scores_out.at[pl.ds(base + row0, K)], osems[buf])
        pltpu.async_copy(rn, nxt_out.at[pl.ds(base + row0, K)], osems[buf])

    for cc in range(2):
        do_chunk(cc, cc)

    def loop_body(i, _):
        for buf in range(2):
            drain(buf)
            do_chunk(2 + 2 * i + buf, buf)
        return 0

    lax.fori_loop(0, (NCHUNK - 2) // 2, loop_body, 0)
    for buf in range(2):
        drain(buf)


@functools.cache
def _build_sc_kernel():
    return pl.kernel(
        _sc_body,
        mesh=plsc.VectorSubcoreMesh(core_axis_name="c", subcore_axis_name="s"),
        compiler_params=pltpu.CompilerParams(
            use_tc_tiling_on_sc=False, needs_layout_passes=False),
        out_type=[
            jax.ShapeDtypeStruct((B, V), jnp.float32),
            jax.ShapeDtypeStruct((B, V), jnp.int32),
        ],
        scratch_types=[
            pltpu.VMEM((LVL * PER_W,), jnp.int32),
            pltpu.VMEM((LVL, PER_W, A), jnp.int32),
            pltpu.VMEM((LVL, PER_W, A), jnp.float32),
            pltpu.VMEM((LVL, PER_W, A), jnp.int32),
            pltpu.VMEM((LVL * PER_W,), jnp.float32),
            pltpu.VMEM(((LVL + 1) * PER_W + 16,), jnp.float32),
            pltpu.VMEM((16,), jnp.float32),
            pltpu.VMEM((2, K, V), jnp.float32),
            pltpu.VMEM((2, K, V), jnp.int32),
            pltpu.SemaphoreType.DMA,
            pltpu.SemaphoreType.DMA,
            pltpu.SemaphoreType.DMA,
        ],
    )


@functools.cache
def _build_t_kernel():
    return pl.kernel(
        _t_body,
        mesh=plsc.VectorSubcoreMesh(core_axis_name="c", subcore_axis_name="s"),
        compiler_params=pltpu.CompilerParams(
            use_tc_tiling_on_sc=False, needs_layout_passes=False),
        out_type=[
            jax.ShapeDtypeStruct((NS_TOT, A), jnp.int32),
            jax.ShapeDtypeStruct((NS_TOT, A), jnp.float32),
            jax.ShapeDtypeStruct((NS_TOT, A), jnp.int32),
        ],
        scratch_types=[
            pltpu.VMEM((2, A, T_CS), jnp.int32),
            pltpu.VMEM((2, A, T_CS), jnp.float32),
            pltpu.VMEM((2, A, T_CS), jnp.int32),
            pltpu.VMEM((2, T_CS, A), jnp.int32),
            pltpu.VMEM((2, T_CS, A), jnp.float32),
            pltpu.VMEM((2, T_CS, A), jnp.int32),
            pltpu.SemaphoreType.DMA,
            pltpu.SemaphoreType.DMA,
            pltpu.SemaphoreType.DMA,
            pltpu.SemaphoreType.DMA,
        ],
    )


def kernel(states, arc_labels, arc_weights, arc_to, backoff_weights,
           backoff_to, unk_prob):
    lbl_r, w_r, to_r = _build_t_kernel()(
        arc_labels.T, arc_weights.T, arc_to.T)
    scores, nxt = _build_sc_kernel()(
        states, lbl_r, w_r, to_r, backoff_weights, backoff_to,
        unk_prob.reshape(1))
    return scores, nxt


# speedup vs baseline: 41.5889x; 1.1248x over previous
import functools

import jax
import jax.numpy as jnp
from jax import lax
from jax.experimental import pallas as pl
from jax.experimental.pallas import tpu as pltpu
from jax.experimental.pallas import tpu_sc as plsc

V = 1024
A = 16
LVL = 4
B = 16384

NW = 32
PER_W = B // NW
PIECE = 128
NPIECE = PER_W // PIECE

K = 8
NCHUNK = PER_W // K

NS_TOT = 250000
T_RANGE = 7816
T_CS = 512
T_NCH = 16


def _t_body(lblT, wT, toT, lbl_out, w_out, to_out,
            pb_l, pb_w, pb_t, rb_l, rb_w, rb_t, gsem0, gsem1, osem0, osem1):
    gsems = (gsem0, gsem1)
    osems = (osem0, osem1)
    nc = plsc.get_sparse_core_info().num_cores
    c = lax.axis_index("c")
    s = lax.axis_index("s")
    wid = s * nc + c
    r0 = jnp.minimum(wid * T_RANGE, NS_TOT - T_RANGE)
    iota16 = lax.iota(jnp.int32, 16)

    def s_of(k):
        return r0 + jnp.minimum(k * T_CS, T_RANGE - T_CS)

    def issue_in(k, par):
        s0 = s_of(k)
        for a in range(16):
            pltpu.async_copy(
                lblT.at[a, pl.ds(s0, T_CS)], pb_l.at[par, a], gsems[par])
            pltpu.async_copy(
                wT.at[a, pl.ds(s0, T_CS)], pb_w.at[par, a], gsems[par])
            pltpu.async_copy(
                toT.at[a, pl.ds(s0, T_CS)], pb_t.at[par, a], gsems[par])

    def wait_in(par):
        for a in range(16):
            pltpu.make_async_copy(
                lblT.at[0, pl.ds(0, T_CS)], pb_l.at[par, a], gsems[par]).wait()
            pltpu.make_async_copy(
                wT.at[0, pl.ds(0, T_CS)], pb_w.at[par, a], gsems[par]).wait()
            pltpu.make_async_copy(
                toT.at[0, pl.ds(0, T_CS)], pb_t.at[par, a], gsems[par]).wait()

    def drain_out(par):
        s0 = s_of(0)
        pltpu.make_async_copy(
            lbl_out.at[pl.ds(s0, T_CS)], rb_l.at[par], osems[par]).wait()
        pltpu.make_async_copy(
            w_out.at[pl.ds(s0, T_CS)], rb_w.at[par], osems[par]).wait()
        pltpu.make_async_copy(
            to_out.at[pl.ds(s0, T_CS)], rb_t.at[par], osems[par]).wait()

    def transpose_and_out(k, par):
        s0 = s_of(k)

        def g_body(g, _):
            ridx = g * 16 + iota16
            for a in range(16):
                cidx = jnp.broadcast_to(jnp.int32(a), (16,))
                plsc.store_scatter(
                    rb_l.at[par], [ridx, cidx],
                    pb_l[par, a, pl.ds(g * 16, 16)])
                plsc.store_scatter(
                    rb_w.at[par], [ridx, cidx],
                    pb_w[par, a, pl.ds(g * 16, 16)])
                plsc.store_scatter(
                    rb_t.at[par], [ridx, cidx],
                    pb_t[par, a, pl.ds(g * 16, 16)])
            return 0

        lax.fori_loop(0, T_CS // 16, g_body, 0)
        pltpu.async_copy(rb_l.at[par], lbl_out.at[pl.ds(s0, T_CS)], osems[par])
        pltpu.async_copy(rb_w.at[par], w_out.at[pl.ds(s0, T_CS)], osems[par])
        pltpu.async_copy(rb_t.at[par], to_out.at[pl.ds(s0, T_CS)], osems[par])

    issue_in(0, 0)
    issue_in(1, 1)
    wait_in(0)
    transpose_and_out(0, 0)
    issue_in(2, 0)
    wait_in(1)
    transpose_and_out(1, 1)
    issue_in(3, 1)

    def pair_body(p, _):
        for par in range(2):
            k = 2 + 2 * p + par
            wait_in(par)
            drain_out(par)
            transpose_and_out(k, par)
            issue_in(k + 2, par)
        return 0

    lax.fori_loop(0, (T_NCH - 4) // 2, pair_body, 0)
    for par in range(2):
        k = T_NCH - 2 + par
        wait_in(par)
        drain_out(par)
        transpose_and_out(k, par)
    for par in range(2):
        drain_out(par)


def _g1_body(states, arc_labels, arc_weights, arc_to, backoff_w,
             backoff_to, lbl_d, w_d, to_d, acc_d,
             cur_v, lbl_v, w_v, to_v, bw_v, acc_v, gsem, osem):
    nc = plsc.get_sparse_core_info().num_cores
    c = lax.axis_index("c")
    s = lax.axis_index("s")
    wid = s * nc + c
    base = wid * PER_W

    pltpu.sync_copy(states.at[pl.ds(base, PER_W)], cur_v.at[pl.ds(0, PER_W)])

    dumps = []
    for l in range(LVL):
        hs = []
        for j in range(NPIECE):
            idx = cur_v.at[pl.ds(l * PER_W + j * PIECE, PIECE)]
            dst = pl.ds(j * PIECE, PIECE)
            hs.append(pltpu.async_copy(arc_labels.at[idx], lbl_v.at[l, dst], gsem))
            hs.append(pltpu.async_copy(arc_weights.at[idx], w_v.at[l, dst], gsem))
            hs.append(pltpu.async_copy(arc_to.at[idx], to_v.at[l, dst], gsem))
            hs.append(pltpu.async_copy(
                backoff_w.at[idx],
                bw_v.at[pl.ds(l * PER_W + j * PIECE, PIECE)], gsem))
            if l + 1 < LVL:
                hs.append(pltpu.async_copy(
                    backoff_to.at[idx],
                    cur_v.at[pl.ds((l + 1) * PER_W + j * PIECE, PIECE)], gsem))
        for h in hs:
            h.wait()
        dumps.append(pltpu.async_copy(
            lbl_v.at[l], lbl_d.at[pl.ds(l * B + base, PER_W)], osem))
        dumps.append(pltpu.async_copy(
            w_v.at[l], w_d.at[pl.ds(l * B + base, PER_W)], osem))
        dumps.append(pltpu.async_copy(
            to_v.at[l], to_d.at[pl.ds(l * B + base, PER_W)], osem))
        for chunk in range(PER_W // 16):
            off = chunk * 16
            if l == 0:
                prev = jnp.zeros((16,), jnp.float32)
                acc_v[pl.ds(off, 16)] = prev
            else:
                prev = acc_v[pl.ds(l * PER_W + off, 16)]
            acc_v[pl.ds((l + 1) * PER_W + off, 16)] = (
                prev + bw_v[pl.ds(l * PER_W + off, 16)])

    for l in range(LVL + 1):
        dumps.append(pltpu.async_copy(
            acc_v.at[pl.ds(l * PER_W, PER_W)],
            acc_d.at[pl.ds(l * B + base, PER_W)], osem))
    for h in dumps:
        h.wait()


def _g2_body(lbl_f, w_f, to_f, acc_f, unk, scores_out, nxt_out,
             lblc0, lblc1, wc0, wc1, toc0, toc1, acc_v, unk_v,
             rs0, rs1, rn0, rn1, csem0, csem1, osem0, osem1):
    nc = plsc.get_sparse_core_info().num_cores
    c = lax.axis_index("c")
    s = lax.axis_index("s")
    wid = s * nc + c
    base = wid * PER_W
    lblcs, wcs, tocs = (lblc0, lblc1), (wc0, wc1), (toc0, toc1)
    rss, rns = (rs0, rs1), (rn0, rn1)
    csems, osems = (csem0, csem1), (osem0, osem1)
    CH = K * A

    for l in range(LVL + 1):
        pltpu.sync_copy(acc_f.at[pl.ds(l * B + base, PER_W)],
                        acc_v.at[pl.ds(l * PER_W, PER_W)])
    pltpu.sync_copy(unk, unk_v.at[pl.ds(0, 1)])
    unk_s = unk_v[...][0]

    def issue_compact(cc, par):
        off = (base + cc * K) * A
        for l in range(LVL):
            pltpu.async_copy(lbl_f.at[pl.ds(l * B * A + off, CH)],
                             lblcs[par].at[pl.ds(l * CH, CH)], csems[par])
            pltpu.async_copy(w_f.at[pl.ds(l * B * A + off, CH)],
                             wcs[par].at[pl.ds(l * CH, CH)], csems[par])
            pltpu.async_copy(to_f.at[pl.ds(l * B * A + off, CH)],
                             tocs[par].at[pl.ds(l * CH, CH)], csems[par])

    def wait_compact(par):
        for l in range(LVL):
            pltpu.make_async_copy(lbl_f.at[pl.ds(0, CH)],
                                  lblcs[par].at[pl.ds(l * CH, CH)],
                                  csems[par]).wait()
            pltpu.make_async_copy(w_f.at[pl.ds(0, CH)],
                                  wcs[par].at[pl.ds(l * CH, CH)],
                                  csems[par]).wait()
            pltpu.make_async_copy(to_f.at[pl.ds(0, CH)],
                                  tocs[par].at[pl.ds(l * CH, CH)],
                                  csems[par]).wait()

    def drain_out(par):
        pltpu.make_async_copy(
            scores_out.at[pl.ds(base, K)], rss[par], osems[par]).wait()
        pltpu.make_async_copy(
            nxt_out.at[pl.ds(base, K)], rns[par], osems[par]).wait()

    def expand(cc, par):
        row0 = cc * K
        rs = rss[par]
        rn = rns[par]
        for r in range(K):
            row = row0 + r
            bval = acc_v[pl.ds(LVL * PER_W + row, 16)][0] + unk_s
            bvec = jnp.broadcast_to(bval, (16,))
            zvec = jnp.zeros((16,), jnp.int32)
            for i in range(V // 16):
                rs[r, pl.ds(i * 16, 16)] = bvec
                rn[r, pl.ds(i * 16, 16)] = zvec
            for l in range(LVL - 1, -1, -1):
                lblv = lblcs[par][pl.ds((l * K + r) * A, 16)]
                wacc = (wcs[par][pl.ds((l * K + r) * A, 16)]
                        + acc_v[pl.ds(l * PER_W + row, 16)][0])
                tov = tocs[par][pl.ds((l * K + r) * A, 16)]
                m = lblv < V
                ridx = jnp.broadcast_to(jnp.int32(r), (16,))
                plsc.store_scatter(rs, [ridx, lblv], wacc, mask=m)
                plsc.store_scatter(rn, [ridx, lblv], tov, mask=m)
        pltpu.async_copy(rs, scores_out.at[pl.ds(base + row0, K)], osems[par])
        pltpu.async_copy(rn, nxt_out.at[pl.ds(base + row0, K)], osems[par])

    issue_compact(0, 0)
    issue_compact(1, 1)

    def pair_body(p, _):
        for par in range(2):
            cc = 2 * p + par
            wait_compact(par)

            @pl.when(p > 0)
            def _():
                drain_out(par)

            expand(cc, par)

            @pl.when(cc + 2 < NCHUNK)
            def _():
                issue_compact(cc + 2, par)
        return 0

    lax.fori_loop(0, NCHUNK // 2, pair_body, 0)
    for par in range(2):
        drain_out(par)


@functools.cache
def _build_g1_kernel():
    return pl.kernel(
        _g1_body,
        mesh=plsc.VectorSubcoreMesh(core_axis_name="c", subcore_axis_name="s"),
        compiler_params=pltpu.CompilerParams(use_tc_tiling_on_sc=False),
        out_type=[
            jax.ShapeDtypeStruct((LVL * B, A), jnp.int32),
            jax.ShapeDtypeStruct((LVL * B, A), jnp.float32),
            jax.ShapeDtypeStruct((LVL * B, A), jnp.int32),
            jax.ShapeDtypeStruct(((LVL + 1) * B,), jnp.float32),
        ],
        scratch_types=[
            pltpu.VMEM((LVL * PER_W,), jnp.int32),
            pltpu.VMEM((LVL, PER_W, A), jnp.int32),
            pltpu.VMEM((LVL, PER_W, A), jnp.float32),
            pltpu.VMEM((LVL, PER_W, A), jnp.int32),
            pltpu.VMEM((LVL * PER_W,), jnp.float32),
            pltpu.VMEM(((LVL + 1) * PER_W + 16,), jnp.float32),
            pltpu.SemaphoreType.DMA,
            pltpu.SemaphoreType.DMA,
        ],
    )


@functools.cache
def _build_g2_kernel():
    return pl.kernel(
        _g2_body,
        mesh=plsc.VectorSubcoreMesh(core_axis_name="c", subcore_axis_name="s"),
        compiler_params=pltpu.CompilerParams(
            use_tc_tiling_on_sc=True, needs_layout_passes=False),
        out_type=[
            jax.ShapeDtypeStruct((B, V), jnp.float32),
            jax.ShapeDtypeStruct((B, V), jnp.int32),
        ],
        scratch_types=[
            pltpu.VMEM((LVL * K * A,), jnp.int32),
            pltpu.VMEM((LVL * K * A,), jnp.int32),
            pltpu.VMEM((LVL * K * A,), jnp.float32),
            pltpu.VMEM((LVL * K * A,), jnp.float32),
            pltpu.VMEM((LVL * K * A,), jnp.int32),
            pltpu.VMEM((LVL * K * A,), jnp.int32),
            pltpu.VMEM(((LVL + 1) * PER_W + 16,), jnp.float32),
            pltpu.VMEM((16,), jnp.float32),
            pltpu.VMEM((K, V), jnp.float32),
            pltpu.VMEM((K, V), jnp.float32),
            pltpu.VMEM((K, V), jnp.int32),
            pltpu.VMEM((K, V), jnp.int32),
            pltpu.SemaphoreType.DMA,
            pltpu.SemaphoreType.DMA,
            pltpu.SemaphoreType.DMA,
            pltpu.SemaphoreType.DMA,
        ],
    )


@functools.cache
def _build_t_kernel():
    return pl.kernel(
        _t_body,
        mesh=plsc.VectorSubcoreMesh(core_axis_name="c", subcore_axis_name="s"),
        compiler_params=pltpu.CompilerParams(
            use_tc_tiling_on_sc=False, needs_layout_passes=False),
        out_type=[
            jax.ShapeDtypeStruct((NS_TOT, A), jnp.int32),
            jax.ShapeDtypeStruct((NS_TOT, A), jnp.float32),
            jax.ShapeDtypeStruct((NS_TOT, A), jnp.int32),
        ],
        scratch_types=[
            pltpu.VMEM((2, A, T_CS), jnp.int32),
            pltpu.VMEM((2, A, T_CS), jnp.float32),
            pltpu.VMEM((2, A, T_CS), jnp.int32),
            pltpu.VMEM((2, T_CS, A), jnp.int32),
            pltpu.VMEM((2, T_CS, A), jnp.float32),
            pltpu.VMEM((2, T_CS, A), jnp.int32),
            pltpu.SemaphoreType.DMA,
            pltpu.SemaphoreType.DMA,
            pltpu.SemaphoreType.DMA,
            pltpu.SemaphoreType.DMA,
        ],
    )


def kernel(states, arc_labels, arc_weights, arc_to, backoff_weights,
           backoff_to, unk_prob):
    lbl_r, w_r, to_r = _build_t_kernel()(
        arc_labels.T, arc_weights.T, arc_to.T)
    lbl_d, w_d, to_d, acc_d = _build_g1_kernel()(
        states, lbl_r, w_r, to_r, backoff_weights, backoff_to)
    scores, nxt = _build_g2_kernel()(
        lbl_d.reshape(-1), w_d.reshape(-1), to_d.reshape(-1), acc_d,
        unk_prob.reshape(1))
    return scores, nxt


# speedup vs baseline: 47.1192x; 1.1330x over previous
import functools

import jax
import jax.numpy as jnp
from jax import lax
from jax.experimental import pallas as pl
from jax.experimental.pallas import tpu as pltpu
from jax.experimental.pallas import tpu_sc as plsc

V = 1024
A = 16
LVL = 4
B = 16384

NW = 32
PER_W = B // NW
PIECE = 128
NPIECE = PER_W // PIECE

K = 8
NCHUNK = PER_W // K
NBUF = 4

NS_TOT = 250000
T_RANGE = 7816
T_CS = 512
T_NCH = 16


def _t_body(lblT, wT, toT, lbl_out, w_out, to_out,
            pb_l, pb_w, pb_t, rb_l, rb_w, rb_t, gsem0, gsem1, osem0, osem1):
    gsems = (gsem0, gsem1)
    osems = (osem0, osem1)
    nc = plsc.get_sparse_core_info().num_cores
    c = lax.axis_index("c")
    s = lax.axis_index("s")
    wid = s * nc + c
    r0 = jnp.minimum(wid * T_RANGE, NS_TOT - T_RANGE)
    iota16 = lax.iota(jnp.int32, 16)

    def s_of(k):
        return r0 + jnp.minimum(k * T_CS, T_RANGE - T_CS)

    def issue_in(k, par):
        s0 = s_of(k)
        for a in range(16):
            pltpu.async_copy(
                lblT.at[a, pl.ds(s0, T_CS)], pb_l.at[par, a], gsems[par])
            pltpu.async_copy(
                wT.at[a, pl.ds(s0, T_CS)], pb_w.at[par, a], gsems[par])
            pltpu.async_copy(
                toT.at[a, pl.ds(s0, T_CS)], pb_t.at[par, a], gsems[par])

    def wait_in(par):
        for a in range(16):
            pltpu.make_async_copy(
                lblT.at[0, pl.ds(0, T_CS)], pb_l.at[par, a], gsems[par]).wait()
            pltpu.make_async_copy(
                wT.at[0, pl.ds(0, T_CS)], pb_w.at[par, a], gsems[par]).wait()
            pltpu.make_async_copy(
                toT.at[0, pl.ds(0, T_CS)], pb_t.at[par, a], gsems[par]).wait()

    def drain_out(par):
        s0 = s_of(0)
        pltpu.make_async_copy(
            lbl_out.at[pl.ds(s0, T_CS)], rb_l.at[par], osems[par]).wait()
        pltpu.make_async_copy(
            w_out.at[pl.ds(s0, T_CS)], rb_w.at[par], osems[par]).wait()
        pltpu.make_async_copy(
            to_out.at[pl.ds(s0, T_CS)], rb_t.at[par], osems[par]).wait()

    def transpose_and_out(k, par):
        s0 = s_of(k)

        def g_body(g, _):
            ridx = g * 16 + iota16
            for pb, rb in ((pb_l, rb_l), (pb_w, rb_w), (pb_t, rb_t)):
                vals = [pb[par, a, pl.ds(g * 16, 16)] for a in range(16)]
                for a in range(16):
                    cidx = jnp.broadcast_to(jnp.int32(a), (16,))
                    plsc.store_scatter(rb.at[par], [ridx, cidx], vals[a])
            return 0

        lax.fori_loop(0, T_CS // 16, g_body, 0)
        pltpu.async_copy(rb_l.at[par], lbl_out.at[pl.ds(s0, T_CS)], osems[par])
        pltpu.async_copy(rb_w.at[par], w_out.at[pl.ds(s0, T_CS)], osems[par])
        pltpu.async_copy(rb_t.at[par], to_out.at[pl.ds(s0, T_CS)], osems[par])

    issue_in(0, 0)
    issue_in(1, 1)
    wait_in(0)
    transpose_and_out(0, 0)
    issue_in(2, 0)
    wait_in(1)
    transpose_and_out(1, 1)
    issue_in(3, 1)

    def pair_body(p, _):
        for par in range(2):
            k = 2 + 2 * p + par
            wait_in(par)
            drain_out(par)
            transpose_and_out(k, par)
            issue_in(k + 2, par)
        return 0

    lax.fori_loop(0, (T_NCH - 4) // 2, pair_body, 0)
    for par in range(2):
        k = T_NCH - 2 + par
        wait_in(par)
        drain_out(par)
        transpose_and_out(k, par)
    for par in range(2):
        drain_out(par)


def _g1_body(states, arc_labels, arc_weights, arc_to, backoff_w,
             backoff_to, lbl_d, w_d, to_d, acc_d,
             cur_v, lbl_v, w_v, to_v, bw_v, acc_v, gsem, osem):
    nc = plsc.get_sparse_core_info().num_cores
    c = lax.axis_index("c")
    s = lax.axis_index("s")
    wid = s * nc + c
    base = wid * PER_W

    pltpu.sync_copy(states.at[pl.ds(base, PER_W)], cur_v.at[pl.ds(0, PER_W)])

    dumps = []
    for l in range(LVL):
        hs = []
        for j in range(NPIECE):
            idx = cur_v.at[pl.ds(l * PER_W + j * PIECE, PIECE)]
            dst = pl.ds(j * PIECE, PIECE)
            hs.append(pltpu.async_copy(arc_labels.at[idx], lbl_v.at[l, dst], gsem))
            hs.append(pltpu.async_copy(arc_weights.at[idx], w_v.at[l, dst], gsem))
            hs.append(pltpu.async_copy(arc_to.at[idx], to_v.at[l, dst], gsem))
            hs.append(pltpu.async_copy(
                backoff_w.at[idx],
                bw_v.at[pl.ds(l * PER_W + j * PIECE, PIECE)], gsem))
            if l + 1 < LVL:
                hs.append(pltpu.async_copy(
                    backoff_to.at[idx],
                    cur_v.at[pl.ds((l + 1) * PER_W + j * PIECE, PIECE)], gsem))
        for h in hs:
            h.wait()
        dumps.append(pltpu.async_copy(
            lbl_v.at[l], lbl_d.at[pl.ds(l * B + base, PER_W)], osem))
        dumps.append(pltpu.async_copy(
            w_v.at[l], w_d.at[pl.ds(l * B + base, PER_W)], osem))
        dumps.append(pltpu.async_copy(
            to_v.at[l], to_d.at[pl.ds(l * B + base, PER_W)], osem))
        for chunk in range(PER_W // 16):
            off = chunk * 16
            if l == 0:
                prev = jnp.zeros((16,), jnp.float32)
                acc_v[pl.ds(off, 16)] = prev
            else:
                prev = acc_v[pl.ds(l * PER_W + off, 16)]
            acc_v[pl.ds((l + 1) * PER_W + off, 16)] = (
                prev + bw_v[pl.ds(l * PER_W + off, 16)])

    for l in range(LVL + 1):
        dumps.append(pltpu.async_copy(
            acc_v.at[pl.ds(l * PER_W, PER_W)],
            acc_d.at[pl.ds(l * B + base, PER_W)], osem))
    for h in dumps:
        h.wait()


def _g2_body(lbl_f, w_f, to_f, acc_f, unk, scores_out, nxt_out,
             lblc0, lblc1, lblc2, lblc3, wc0, wc1, wc2, wc3,
             toc0, toc1, toc2, toc3, acc_v, unk_v, rs_v, rn_v,
             csem0, csem1, csem2, csem3, osem0, osem1, osem2, osem3):
    nc = plsc.get_sparse_core_info().num_cores
    c = lax.axis_index("c")
    s = lax.axis_index("s")
    wid = s * nc + c
    base = wid * PER_W
    lblcs = (lblc0, lblc1, lblc2, lblc3)
    wcs = (wc0, wc1, wc2, wc3)
    tocs = (toc0, toc1, toc2, toc3)
    rss = tuple(rs_v.at[b] for b in range(NBUF))
    rns = tuple(rn_v.at[b] for b in range(NBUF))
    csems = (csem0, csem1, csem2, csem3)
    osems = (osem0, osem1, osem2, osem3)
    CH = K * A

    for l in range(LVL + 1):
        pltpu.sync_copy(acc_f.at[pl.ds(l * B + base, PER_W)],
                        acc_v.at[pl.ds(l * PER_W, PER_W)])
    pltpu.sync_copy(unk, unk_v.at[pl.ds(0, 1)])
    unk_s = unk_v[...][0]

    def issue_compact(cc, par):
        off = (base + cc * K) * A
        for l in range(LVL):
            pltpu.async_copy(lbl_f.at[pl.ds(l * B * A + off, CH)],
                             lblcs[par].at[pl.ds(l * CH, CH)], csems[par])
            pltpu.async_copy(w_f.at[pl.ds(l * B * A + off, CH)],
                             wcs[par].at[pl.ds(l * CH, CH)], csems[par])
            pltpu.async_copy(to_f.at[pl.ds(l * B * A + off, CH)],
                             tocs[par].at[pl.ds(l * CH, CH)], csems[par])

    def wait_compact(par):
        for l in range(LVL):
            pltpu.make_async_copy(lbl_f.at[pl.ds(0, CH)],
                                  lblcs[par].at[pl.ds(l * CH, CH)],
                                  csems[par]).wait()
            pltpu.make_async_copy(w_f.at[pl.ds(0, CH)],
                                  wcs[par].at[pl.ds(l * CH, CH)],
                                  csems[par]).wait()
            pltpu.make_async_copy(to_f.at[pl.ds(0, CH)],
                                  tocs[par].at[pl.ds(l * CH, CH)],
                                  csems[par]).wait()

    def drain_out(par):
        pltpu.make_async_copy(
            scores_out.at[pl.ds(base, K)], rss[par], osems[par]).wait()
        pltpu.make_async_copy(
            nxt_out.at[pl.ds(base, K)], rns[par], osems[par]).wait()

    def expand(cc, par):
        row0 = cc * K
        rs = rss[par]
        rn = rns[par]
        for r in range(K):
            row = row0 + r
            bval = acc_v[pl.ds(LVL * PER_W + row, 16)][0] + unk_s
            bvec = jnp.broadcast_to(bval, (16,))
            zvec = jnp.zeros((16,), jnp.int32)
            for i in range(V // 16):
                rs[r, pl.ds(i * 16, 16)] = bvec
                rn[r, pl.ds(i * 16, 16)] = zvec
            for l in range(LVL - 1, -1, -1):
                lblv = lblcs[par][pl.ds((l * K + r) * A, 16)]
                wacc = (wcs[par][pl.ds((l * K + r) * A, 16)]
                        + acc_v[pl.ds(l * PER_W + row, 16)][0])
                tov = tocs[par][pl.ds((l * K + r) * A, 16)]
                m = lblv < V
                ridx = jnp.broadcast_to(jnp.int32(r), (16,))
                plsc.store_scatter(rs, [ridx, lblv], wacc, mask=m)
                plsc.store_scatter(rn, [ridx, lblv], tov, mask=m)
        pltpu.async_copy(rs, scores_out.at[pl.ds(base + row0, K)], osems[par])
        pltpu.async_copy(rn, nxt_out.at[pl.ds(base + row0, K)], osems[par])

    for b in range(NBUF):
        issue_compact(b, b)

    def pair_body(p, _):
        for par in range(NBUF):
            cc = NBUF * p + par
            wait_compact(par)

            @pl.when(p > 0)
            def _():
                drain_out(par)

            expand(cc, par)

            @pl.when(cc + NBUF < NCHUNK)
            def _():
                issue_compact(cc + NBUF, par)
        return 0

    lax.fori_loop(0, NCHUNK // NBUF, pair_body, 0)
    for par in range(NBUF):
        drain_out(par)


@functools.cache
def _build_g1_kernel():
    return pl.kernel(
        _g1_body,
        mesh=plsc.VectorSubcoreMesh(core_axis_name="c", subcore_axis_name="s"),
        compiler_params=pltpu.CompilerParams(use_tc_tiling_on_sc=False),
        out_type=[
            jax.ShapeDtypeStruct((LVL * B, A), jnp.int32),
            jax.ShapeDtypeStruct((LVL * B, A), jnp.float32),
            jax.ShapeDtypeStruct((LVL * B, A), jnp.int32),
            jax.ShapeDtypeStruct(((LVL + 1) * B,), jnp.float32),
        ],
        scratch_types=[
            pltpu.VMEM((LVL * PER_W,), jnp.int32),
            pltpu.VMEM((LVL, PER_W, A), jnp.int32),
            pltpu.VMEM((LVL, PER_W, A), jnp.float32),
            pltpu.VMEM((LVL, PER_W, A), jnp.int32),
            pltpu.VMEM((LVL * PER_W,), jnp.float32),
            pltpu.VMEM(((LVL + 1) * PER_W + 16,), jnp.float32),
            pltpu.SemaphoreType.DMA,
            pltpu.SemaphoreType.DMA,
        ],
    )


@functools.cache
def _build_g2_kernel():
    return pl.kernel(
        _g2_body,
        mesh=plsc.VectorSubcoreMesh(core_axis_name="c", subcore_axis_name="s"),
        compiler_params=pltpu.CompilerParams(
            use_tc_tiling_on_sc=True, needs_layout_passes=False),
        out_type=[
            jax.ShapeDtypeStruct((B, V), jnp.float32),
            jax.ShapeDtypeStruct((B, V), jnp.int32),
        ],
        scratch_types=(
            [pltpu.VMEM((LVL * K * A,), jnp.int32) for _ in range(NBUF)]
            + [pltpu.VMEM((LVL * K * A,), jnp.float32) for _ in range(NBUF)]
            + [pltpu.VMEM((LVL * K * A,), jnp.int32) for _ in range(NBUF)]
            + [
                pltpu.VMEM(((LVL + 1) * PER_W + 16,), jnp.float32),
                pltpu.VMEM((16,), jnp.float32),
                pltpu.VMEM((NBUF, K, V), jnp.float32),
                pltpu.VMEM((NBUF, K, V), jnp.int32),
            ]
            + [pltpu.SemaphoreType.DMA for _ in range(2 * NBUF)]
        ),
    )


@functools.cache
def _build_t_kernel():
    return pl.kernel(
        _t_body,
        mesh=plsc.VectorSubcoreMesh(core_axis_name="c", subcore_axis_name="s"),
        compiler_params=pltpu.CompilerParams(
            use_tc_tiling_on_sc=False, needs_layout_passes=False),
        out_type=[
            jax.ShapeDtypeStruct((NS_TOT, A), jnp.int32),
            jax.ShapeDtypeStruct((NS_TOT, A), jnp.float32),
            jax.ShapeDtypeStruct((NS_TOT, A), jnp.int32),
        ],
        scratch_types=[
            pltpu.VMEM((2, A, T_CS), jnp.int32),
            pltpu.VMEM((2, A, T_CS), jnp.float32),
            pltpu.VMEM((2, A, T_CS), jnp.int32),
            pltpu.VMEM((2, T_CS, A), jnp.int32),
            pltpu.VMEM((2, T_CS, A), jnp.float32),
            pltpu.VMEM((2, T_CS, A), jnp.int32),
            pltpu.SemaphoreType.DMA,
            pltpu.SemaphoreType.DMA,
            pltpu.SemaphoreType.DMA,
            pltpu.SemaphoreType.DMA,
        ],
    )


def kernel(states, arc_labels, arc_weights, arc_to, backoff_weights,
           backoff_to, unk_prob):
    lbl_r, w_r, to_r = _build_t_kernel()(
        arc_labels.T, arc_weights.T, arc_to.T)
    lbl_d, w_d, to_d, acc_d = _build_g1_kernel()(
        states, lbl_r, w_r, to_r, backoff_weights, backoff_to)
    scores, nxt = _build_g2_kernel()(
        lbl_d.reshape(-1), w_d.reshape(-1), to_d.reshape(-1), acc_d,
        unk_prob.reshape(1))
    return scores, nxt


# speedup vs baseline: 52.9111x; 1.1229x over previous
import functools

import jax
import jax.numpy as jnp
from jax import lax
from jax.experimental import pallas as pl
from jax.experimental.pallas import tpu as pltpu
from jax.experimental.pallas import tpu_sc as plsc

V = 1024
A = 16
LVL = 4
B = 16384

NW = 32
PER_W = B // NW
PIECE = 128
NPIECE = PER_W // PIECE

K = 8
NCHUNK = PER_W // K
NBUF = 2

NS_TOT = 250000
T_RANGE = 7816
T_CS = 512
T_NCH = 16


def _t_body(lblT, wT, toT, lbl_out, w_out, to_out,
            pb_l, pb_w, pb_t, rb_l, rb_w, rb_t, gsem0, gsem1, osem0, osem1):
    gsems = (gsem0, gsem1)
    osems = (osem0, osem1)
    nc = plsc.get_sparse_core_info().num_cores
    c = lax.axis_index("c")
    s = lax.axis_index("s")
    wid = s * nc + c
    r0 = jnp.minimum(wid * T_RANGE, NS_TOT - T_RANGE)
    iota16 = lax.iota(jnp.int32, 16)

    def s_of(k):
        return r0 + jnp.minimum(k * T_CS, T_RANGE - T_CS)

    def issue_in(k, par):
        s0 = s_of(k)
        for a in range(16):
            pltpu.async_copy(
                lblT.at[a, pl.ds(s0, T_CS)], pb_l.at[par, a], gsems[par])
            pltpu.async_copy(
                wT.at[a, pl.ds(s0, T_CS)], pb_w.at[par, a], gsems[par])
            pltpu.async_copy(
                toT.at[a, pl.ds(s0, T_CS)], pb_t.at[par, a], gsems[par])

    def wait_in(par):
        for a in range(16):
            pltpu.make_async_copy(
                lblT.at[0, pl.ds(0, T_CS)], pb_l.at[par, a], gsems[par]).wait()
            pltpu.make_async_copy(
                wT.at[0, pl.ds(0, T_CS)], pb_w.at[par, a], gsems[par]).wait()
            pltpu.make_async_copy(
                toT.at[0, pl.ds(0, T_CS)], pb_t.at[par, a], gsems[par]).wait()

    def drain_out(par):
        s0 = s_of(0)
        pltpu.make_async_copy(
            lbl_out.at[pl.ds(s0, T_CS)], rb_l.at[par], osems[par]).wait()
        pltpu.make_async_copy(
            w_out.at[pl.ds(s0, T_CS)], rb_w.at[par], osems[par]).wait()
        pltpu.make_async_copy(
            to_out.at[pl.ds(s0, T_CS)], rb_t.at[par], osems[par]).wait()

    def transpose_and_out(k, par):
        s0 = s_of(k)

        def g_body(g, _):
            ridx = g * 16 + iota16
            for pb, rb in ((pb_l, rb_l), (pb_w, rb_w), (pb_t, rb_t)):
                vals = [pb[par, a, pl.ds(g * 16, 16)] for a in range(16)]
                for a in range(16):
                    cidx = jnp.broadcast_to(jnp.int32(a), (16,))
                    plsc.store_scatter(rb.at[par], [ridx, cidx], vals[a])
            return 0

        lax.fori_loop(0, T_CS // 16, g_body, 0)
        pltpu.async_copy(rb_l.at[par], lbl_out.at[pl.ds(s0, T_CS)], osems[par])
        pltpu.async_copy(rb_w.at[par], w_out.at[pl.ds(s0, T_CS)], osems[par])
        pltpu.async_copy(rb_t.at[par], to_out.at[pl.ds(s0, T_CS)], osems[par])

    issue_in(0, 0)
    issue_in(1, 1)
    wait_in(0)
    transpose_and_out(0, 0)
    issue_in(2, 0)
    wait_in(1)
    transpose_and_out(1, 1)
    issue_in(3, 1)

    def pair_body(p, _):
        for par in range(2):
            k = 2 + 2 * p + par
            wait_in(par)
            drain_out(par)
            transpose_and_out(k, par)
            issue_in(k + 2, par)
        return 0

    lax.fori_loop(0, (T_NCH - 4) // 2, pair_body, 0)
    for par in range(2):
        k = T_NCH - 2 + par
        wait_in(par)
        drain_out(par)
        transpose_and_out(k, par)
    for par in range(2):
        drain_out(par)


def _g1_body(states, arc_labels, arc_weights, arc_to, backoff_w,
             backoff_to, lbl_d, w_d, to_d, acc_d,
             cur_v, lbl_v, w_v, to_v, bw_v, acc_v, gsem, osem):
    nc = plsc.get_sparse_core_info().num_cores
    c = lax.axis_index("c")
    s = lax.axis_index("s")
    wid = s * nc + c
    base = wid * PER_W

    pltpu.sync_copy(states.at[pl.ds(base, PER_W)], cur_v.at[pl.ds(0, PER_W)])

    dumps = []
    for l in range(LVL):
        hs = []
        for j in range(NPIECE):
            idx = cur_v.at[pl.ds(l * PER_W + j * PIECE, PIECE)]
            dst = pl.ds(j * PIECE, PIECE)
            hs.append(pltpu.async_copy(arc_labels.at[idx], lbl_v.at[l, dst], gsem))
            hs.append(pltpu.async_copy(arc_weights.at[idx], w_v.at[l, dst], gsem))
            hs.append(pltpu.async_copy(arc_to.at[idx], to_v.at[l, dst], gsem))
            hs.append(pltpu.async_copy(
                backoff_w.at[idx],
                bw_v.at[pl.ds(l * PER_W + j * PIECE, PIECE)], gsem))
            if l + 1 < LVL:
                hs.append(pltpu.async_copy(
                    backoff_to.at[idx],
                    cur_v.at[pl.ds((l + 1) * PER_W + j * PIECE, PIECE)], gsem))
        for h in hs:
            h.wait()
        dumps.append(pltpu.async_copy(
            lbl_v.at[l], lbl_d.at[pl.ds(l * B + base, PER_W)], osem))
        dumps.append(pltpu.async_copy(
            w_v.at[l], w_d.at[pl.ds(l * B + base, PER_W)], osem))
        dumps.append(pltpu.async_copy(
            to_v.at[l], to_d.at[pl.ds(l * B + base, PER_W)], osem))
        for chunk in range(PER_W // 16):
            off = chunk * 16
            if l == 0:
                prev = jnp.zeros((16,), jnp.float32)
                acc_v[pl.ds(off, 16)] = prev
            else:
                prev = acc_v[pl.ds(l * PER_W + off, 16)]
            acc_v[pl.ds((l + 1) * PER_W + off, 16)] = (
                prev + bw_v[pl.ds(l * PER_W + off, 16)])

    for l in range(LVL + 1):
        dumps.append(pltpu.async_copy(
            acc_v.at[pl.ds(l * PER_W, PER_W)],
            acc_d.at[pl.ds(l * B + base, PER_W)], osem))
    for h in dumps:
        h.wait()


def _g2_body(lbl_f, w_f, to_f, acc_f, unk, scores_out, nxt_out, *sc):
    nc = plsc.get_sparse_core_info().num_cores
    c = lax.axis_index("c")
    s = lax.axis_index("s")
    wid = s * nc + c
    base = wid * PER_W
    lblcs = sc[0:NBUF]
    wcs = sc[NBUF:2 * NBUF]
    tocs = sc[2 * NBUF:3 * NBUF]
    acc_v, unk_v, rs_v, rn_v = sc[3 * NBUF:3 * NBUF + 4]
    csems = sc[3 * NBUF + 4:3 * NBUF + 4 + NBUF]
    osems = sc[3 * NBUF + 4 + NBUF:]
    rss = tuple(rs_v.at[b] for b in range(NBUF))
    rns = tuple(rn_v.at[b] for b in range(NBUF))
    CH = K * A

    for l in range(LVL + 1):
        pltpu.sync_copy(acc_f.at[pl.ds(l * B + base, PER_W)],
                        acc_v.at[pl.ds(l * PER_W, PER_W)])
    pltpu.sync_copy(unk, unk_v.at[pl.ds(0, 1)])
    unk_s = unk_v[...][0]

    def issue_compact(cc, par):
        off = (base + cc * K) * A
        for l in range(LVL):
            pltpu.async_copy(lbl_f.at[pl.ds(l * B * A + off, CH)],
                             lblcs[par].at[pl.ds(l * CH, CH)], csems[par])
            pltpu.async_copy(w_f.at[pl.ds(l * B * A + off, CH)],
                             wcs[par].at[pl.ds(l * CH, CH)], csems[par])
            pltpu.async_copy(to_f.at[pl.ds(l * B * A + off, CH)],
                             tocs[par].at[pl.ds(l * CH, CH)], csems[par])

    def wait_compact(par):
        for l in range(LVL):
            pltpu.make_async_copy(lbl_f.at[pl.ds(0, CH)],
                                  lblcs[par].at[pl.ds(l * CH, CH)],
                                  csems[par]).wait()
            pltpu.make_async_copy(w_f.at[pl.ds(0, CH)],
                                  wcs[par].at[pl.ds(l * CH, CH)],
                                  csems[par]).wait()
            pltpu.make_async_copy(to_f.at[pl.ds(0, CH)],
                                  tocs[par].at[pl.ds(l * CH, CH)],
                                  csems[par]).wait()

    def drain_out(par):
        pltpu.make_async_copy(
            scores_out.at[pl.ds(base, K)], rss[par], osems[par]).wait()
        pltpu.make_async_copy(
            nxt_out.at[pl.ds(base, K)], rns[par], osems[par]).wait()

    def expand(cc, par):
        row0 = cc * K
        rs = rss[par]
        rn = rns[par]
        for r in range(K):
            row = row0 + r
            bval = acc_v[pl.ds(LVL * PER_W + row, 16)][0] + unk_s
            bvec = jnp.broadcast_to(bval, (16,))
            zvec = jnp.zeros((16,), jnp.int32)
            for i in range(V // 16):
                rs[r, pl.ds(i * 16, 16)] = bvec
                rn[r, pl.ds(i * 16, 16)] = zvec
            for l in range(LVL - 1, -1, -1):
                lblv = lblcs[par][pl.ds((l * K + r) * A, 16)]
                wacc = (wcs[par][pl.ds((l * K + r) * A, 16)]
                        + acc_v[pl.ds(l * PER_W + row, 16)][0])
                tov = tocs[par][pl.ds((l * K + r) * A, 16)]
                m = lblv < V
                ridx = jnp.broadcast_to(jnp.int32(r), (16,))
                plsc.store_scatter(rs, [ridx, lblv], wacc, mask=m)
                plsc.store_scatter(rn, [ridx, lblv], tov, mask=m)
        pltpu.async_copy(rs, scores_out.at[pl.ds(base + row0, K)], osems[par])
        pltpu.async_copy(rn, nxt_out.at[pl.ds(base + row0, K)], osems[par])

    for b in range(NBUF):
        issue_compact(b, b)

    def pair_body(p, _):
        for par in range(NBUF):
            cc = NBUF * p + par
            wait_compact(par)

            @pl.when(p > 0)
            def _():
                drain_out(par)

            expand(cc, par)

            @pl.when(cc + NBUF < NCHUNK)
            def _():
                issue_compact(cc + NBUF, par)
        return 0

    lax.fori_loop(0, NCHUNK // NBUF, pair_body, 0)
    for par in range(NBUF):
        drain_out(par)


@functools.cache
def _build_g1_kernel():
    return pl.kernel(
        _g1_body,
        mesh=plsc.VectorSubcoreMesh(core_axis_name="c", subcore_axis_name="s"),
        compiler_params=pltpu.CompilerParams(use_tc_tiling_on_sc=False),
        out_type=[
            jax.ShapeDtypeStruct((LVL * B, A), jnp.int32),
            jax.ShapeDtypeStruct((LVL * B, A), jnp.float32),
            jax.ShapeDtypeStruct((LVL * B, A), jnp.int32),
            jax.ShapeDtypeStruct(((LVL + 1) * B,), jnp.float32),
        ],
        scratch_types=[
            pltpu.VMEM((LVL * PER_W,), jnp.int32),
            pltpu.VMEM((LVL, PER_W, A), jnp.int32),
            pltpu.VMEM((LVL, PER_W, A), jnp.float32),
            pltpu.VMEM((LVL, PER_W, A), jnp.int32),
            pltpu.VMEM((LVL * PER_W,), jnp.float32),
            pltpu.VMEM(((LVL + 1) * PER_W + 16,), jnp.float32),
            pltpu.SemaphoreType.DMA,
            pltpu.SemaphoreType.DMA,
        ],
    )


@functools.cache
def _build_g2_kernel():
    return pl.kernel(
        _g2_body,
        mesh=plsc.VectorSubcoreMesh(core_axis_name="c", subcore_axis_name="s"),
        compiler_params=pltpu.CompilerParams(
            use_tc_tiling_on_sc=True, needs_layout_passes=False),
        out_type=[
            jax.ShapeDtypeStruct((B, V), jnp.float32),
            jax.ShapeDtypeStruct((B, V), jnp.int32),
        ],
        scratch_types=(
            [pltpu.VMEM((LVL * K * A,), jnp.int32) for _ in range(NBUF)]
            + [pltpu.VMEM((LVL * K * A,), jnp.float32) for _ in range(NBUF)]
            + [pltpu.VMEM((LVL * K * A,), jnp.int32) for _ in range(NBUF)]
            + [
                pltpu.VMEM(((LVL + 1) * PER_W + 16,), jnp.float32),
                pltpu.VMEM((16,), jnp.float32),
                pltpu.VMEM((NBUF, K, V), jnp.float32),
                pltpu.VMEM((NBUF, K, V), jnp.int32),
            ]
            + [pltpu.SemaphoreType.DMA for _ in range(2 * NBUF)]
        ),
    )


@functools.cache
def _build_t_kernel():
    return pl.kernel(
        _t_body,
        mesh=plsc.VectorSubcoreMesh(core_axis_name="c", subcore_axis_name="s"),
        compiler_params=pltpu.CompilerParams(
            use_tc_tiling_on_sc=False, needs_layout_passes=False),
        out_type=[
            jax.ShapeDtypeStruct((NS_TOT, A), jnp.int32),
            jax.ShapeDtypeStruct((NS_TOT, A), jnp.float32),
            jax.ShapeDtypeStruct((NS_TOT, A), jnp.int32),
        ],
        scratch_types=[
            pltpu.VMEM((2, A, T_CS), jnp.int32),
            pltpu.VMEM((2, A, T_CS), jnp.float32),
            pltpu.VMEM((2, A, T_CS), jnp.int32),
            pltpu.VMEM((2, T_CS, A), jnp.int32),
            pltpu.VMEM((2, T_CS, A), jnp.float32),
            pltpu.VMEM((2, T_CS, A), jnp.int32),
            pltpu.SemaphoreType.DMA,
            pltpu.SemaphoreType.DMA,
            pltpu.SemaphoreType.DMA,
            pltpu.SemaphoreType.DMA,
        ],
    )


def kernel(states, arc_labels, arc_weights, arc_to, backoff_weights,
           backoff_to, unk_prob):
    lbl_r, w_r, to_r = _build_t_kernel()(
        arc_labels.T, arc_weights.T, arc_to.T)
    lbl_d, w_d, to_d, acc_d = _build_g1_kernel()(
        states, lbl_r, w_r, to_r, backoff_weights, backoff_to)
    scores, nxt = _build_g2_kernel()(
        lbl_d.reshape(-1), w_d.reshape(-1), to_d.reshape(-1), acc_d,
        unk_prob.reshape(1))
    return scores, nxt
